# baseline (speedup 1.0000x reference)
import functools

import jax
import jax.numpy as jnp
from jax import lax
from jax.experimental import pallas as pl
from jax.experimental.pallas import tpu as pltpu
from jax.experimental.pallas import tpu_sc as plsc

N_NODES = 10000
D_FEAT = 128
N_HID = 128
N_EDGES = 320000

NC = 2
NS = 16
NW = NC * NS
K = 128
NCHT = N_EDGES // K
CHB = NCHT // NW
CHX = NCHT - CHB * NW
NROWS = 10112
RPT = NROWS // NS
NPAD = 16384
DPT = NPAD // NS

_mesh = plsc.VectorSubcoreMesh(core_axis_name="c", subcore_axis_name="s")


def _chunk_range(c, t):
    w = c * NS + t
    start = w * CHB + jnp.minimum(w, CHX)
    nch = CHB + jnp.where(w < CHX, 1, 0)
    return start, nch


@functools.partial(
    pl.kernel,
    out_type=jax.ShapeDtypeStruct((NC * NPAD,), jnp.float32),
    mesh=_mesh,
    scratch_types=dict(
        deg_sh=pltpu.VMEM_SHARED((NPAD,), jnp.float32),
        dstb=pltpu.VMEM((CHB, K), jnp.int32),
        xtra=pltpu.VMEM((K,), jnp.int32),
        ones=pltpu.VMEM((K,), jnp.float32),
        zv=pltpu.VMEM((DPT,), jnp.float32),
        semd=pltpu.SemaphoreType.DMA,
    ),
)
def _sc_deg(edge_hbm, deg_out, *, deg_sh, dstb, xtra, ones, zv, semd):
    c = lax.axis_index("c")
    t = lax.axis_index("s")
    start, nch = _chunk_range(c, t)

    def stage(j, carry):
        off = pl.multiple_of(N_EDGES + (start + j) * K, 8)
        pltpu.async_copy(edge_hbm.at[pl.ds(off, K)], dstb.at[j], semd)
        return carry

    lax.fori_loop(0, CHB, stage, 0)

    @pl.when(nch > CHB)
    def _():
        pltpu.sync_copy(
            edge_hbm.at[pl.ds(pl.multiple_of(N_EDGES + (start + CHB) * K, 8), K)],
            xtra)

    def drain(j, carry):
        pltpu.make_async_copy(edge_hbm.at[pl.ds(0, K)], dstb.at[0], semd).wait()
        return carry

    lax.fori_loop(0, CHB, drain, 0)

    for i in range(K // 16):
        ones[pl.ds(i * 16, 16)] = jnp.ones((16,), jnp.float32)
    for i in range(DPT // 16):
        zv[pl.ds(i * 16, 16)] = jnp.zeros((16,), jnp.float32)
    pltpu.sync_copy(zv, deg_sh.at[pl.ds(t * DPT, DPT)])
    plsc.subcore_barrier()

    def chunk(j, carry):
        pltpu.sync_copy(ones, deg_sh.at[dstb.at[j]], add=True)
        return carry

    lax.fori_loop(0, CHB, chunk, 0)

    @pl.when(nch > CHB)
    def _():
        pltpu.sync_copy(ones, deg_sh.at[xtra], add=True)

    plsc.subcore_barrier()
    pltpu.sync_copy(
        deg_sh.at[pl.ds(t * DPT, DPT)],
        deg_out.at[pl.ds(c * NPAD + t * DPT, DPT)],
    )


@functools.partial(
    pl.kernel,
    out_type=jax.ShapeDtypeStruct((NC, NROWS, N_HID), jnp.float32),
    mesh=_mesh,
    scratch_types=dict(
        acc_sh=pltpu.VMEM_SHARED((NROWS, N_HID), jnp.float32),
        ib0s=pltpu.VMEM((K,), jnp.int32),
        ib1s=pltpu.VMEM((K,), jnp.int32),
        ib0d=pltpu.VMEM((K,), jnp.int32),
        ib1d=pltpu.VMEM((K,), jnp.int32),
        rows0=pltpu.VMEM((K, N_HID), jnp.float32),
        rows1=pltpu.VMEM((K, N_HID), jnp.float32),
        sem0=pltpu.SemaphoreType.DMA,
        sem1=pltpu.SemaphoreType.DMA,
        sem0b=pltpu.SemaphoreType.DMA,
        sem1b=pltpu.SemaphoreType.DMA,
        semi0s=pltpu.SemaphoreType.DMA,
        semi1s=pltpu.SemaphoreType.DMA,
        semi0d=pltpu.SemaphoreType.DMA,
        semi1d=pltpu.SemaphoreType.DMA,
    ),
)
def _sc_scatter(edge_hbm, g_hbm, out_hbm, *, acc_sh, ib0s, ib1s, ib0d, ib1d,
                rows0, rows1, sem0, sem1, sem0b, sem1b,
                semi0s, semi1s, semi0d, semi1d):

    def _ig(ib, rows, sa, sb):
        pltpu.async_copy(g_hbm.at[ib.at[pl.ds(0, K // 2)]],
                         rows.at[pl.ds(0, K // 2)], sa)
        pltpu.async_copy(g_hbm.at[ib.at[pl.ds(K // 2, K // 2)]],
                         rows.at[pl.ds(K // 2, K // 2)], sb)

    def _wg(ib, rows, sa, sb):
        pltpu.make_async_copy(g_hbm.at[ib.at[pl.ds(0, K // 2)]],
                              rows.at[pl.ds(0, K // 2)], sa).wait()
        pltpu.make_async_copy(g_hbm.at[ib.at[pl.ds(0, K // 2)]],
                              rows.at[pl.ds(K // 2, K // 2)], sb).wait()
    c = lax.axis_index("c")
    t = lax.axis_index("s")
    start, nch = _chunk_range(c, t)

    def src_at(j):
        return edge_hbm.at[pl.ds(pl.multiple_of((start + j) * K, 8), K)]

    def dst_at(j):
        return edge_hbm.at[pl.ds(pl.multiple_of(N_EDGES + (start + j) * K, 8), K)]

    def zrow(i, carry):
        for j in range(N_HID // 16):
            rows0[i, pl.ds(j * 16, 16)] = jnp.zeros((16,), jnp.float32)
        return carry

    lax.fori_loop(0, K, zrow, 0)
    nz = RPT // K
    tail = RPT - nz * K
    for i in range(nz):
        pltpu.async_copy(rows0, acc_sh.at[pl.ds(t * RPT + i * K, K)], sem0)
    pltpu.async_copy(rows0.at[pl.ds(0, tail)],
                     acc_sh.at[pl.ds(t * RPT + nz * K, tail)], sem1)
    for _ in range(nz):
        pltpu.make_async_copy(rows0, acc_sh.at[pl.ds(t * RPT, K)], sem0).wait()
    pltpu.make_async_copy(rows0.at[pl.ds(0, tail)],
                          acc_sh.at[pl.ds(t * RPT, tail)], sem1).wait()
    plsc.subcore_barrier()

    pltpu.async_copy(src_at(0), ib0s, semi0s)
    pltpu.async_copy(src_at(1), ib1s, semi1s)
    pltpu.async_copy(dst_at(0), ib0d, semi0d)
    pltpu.async_copy(dst_at(1), ib1d, semi1d)
    pltpu.make_async_copy(src_at(0), ib0s, semi0s).wait()
    _ig(ib0s, rows0, sem0, sem0b)
    pltpu.make_async_copy(src_at(1), ib1s, semi1s).wait()
    _ig(ib1s, rows1, sem1, sem1b)

    def pair(i, carry):
        j0 = 2 * i
        _wg(ib0s, rows0, sem0, sem0b)
        pltpu.make_async_copy(dst_at(0), ib0d, semi0d).wait()

        @pl.when(j0 + 2 < nch)
        def _():
            pltpu.async_copy(src_at(j0 + 2), ib0s, semi0s)

        pltpu.sync_copy(rows0, acc_sh.at[ib0d], add=True)

        @pl.when(j0 + 2 < nch)
        def _():
            pltpu.async_copy(dst_at(j0 + 2), ib0d, semi0d)
            pltpu.make_async_copy(src_at(0), ib0s, semi0s).wait()
            _ig(ib0s, rows0, sem0, sem0b)

        _wg(ib1s, rows1, sem1, sem1b)
        pltpu.make_async_copy(dst_at(1), ib1d, semi1d).wait()

        @pl.when(j0 + 3 < nch)
        def _():
            pltpu.async_copy(src_at(j0 + 3), ib1s, semi1s)

        pltpu.sync_copy(rows1, acc_sh.at[ib1d], add=True)

        @pl.when(j0 + 3 < nch)
        def _():
            pltpu.async_copy(dst_at(j0 + 3), ib1d, semi1d)
            pltpu.make_async_copy(src_at(0), ib1s, semi1s).wait()
            _ig(ib1s, rows1, sem1, sem1b)

        return carry

    lax.fori_loop(0, CHB // 2, pair, 0)

    @pl.when(nch > CHB)
    def _():
        _wg(ib0s, rows0, sem0, sem0b)
        pltpu.make_async_copy(dst_at(0), ib0d, semi0d).wait()
        pltpu.sync_copy(rows0, acc_sh.at[ib0d], add=True)

    plsc.subcore_barrier()
    pltpu.sync_copy(
        acc_sh.at[pl.ds(t * RPT, RPT)],
        out_hbm.at[c, pl.ds(t * RPT, RPT)],
    )


_RB = 2000
_GRID = N_NODES // _RB


def _tc_mm_body(x_ref, w_ref, h_ref):
    h_ref[...] = jnp.dot(x_ref[...], w_ref[...],
                         preferred_element_type=jnp.float32)


def _tc_mm(x, W1):
    return pl.pallas_call(
        _tc_mm_body,
        grid=(_GRID,),
        in_specs=[
            pl.BlockSpec((_RB, D_FEAT), lambda i: (i, 0)),
            pl.BlockSpec((D_FEAT, N_HID), lambda i: (0, 0)),
        ],
        out_specs=pl.BlockSpec((_RB, N_HID), lambda i: (i, 0)),
        out_shape=jax.ShapeDtypeStruct((N_NODES, N_HID), jnp.float32),
    )(x, W1)


def _tc_scale_body(h_ref, d0_ref, d1_ref, g_ref, dinv_ref):
    deg = d0_ref[...] + d1_ref[...] + 1.0
    dv = lax.rsqrt(deg)
    dinv_ref[...] = dv
    g_ref[...] = h_ref[...] * dv


def _tc_scale(h, deg0, deg1):
    return pl.pallas_call(
        _tc_scale_body,
        grid=(_GRID,),
        in_specs=[
            pl.BlockSpec((_RB, N_HID), lambda i: (i, 0)),
            pl.BlockSpec((_RB, 1), lambda i: (i, 0)),
            pl.BlockSpec((_RB, 1), lambda i: (i, 0)),
        ],
        out_specs=[
            pl.BlockSpec((_RB, N_HID), lambda i: (i, 0)),
            pl.BlockSpec((_RB, 1), lambda i: (i, 0)),
        ],
        out_shape=[
            jax.ShapeDtypeStruct((N_NODES, N_HID), jnp.float32),
            jax.ShapeDtypeStruct((N_NODES, 1), jnp.float32),
        ],
    )(h, deg0, deg1)


def _tc_mid_body(s_ref, g_ref, dv_ref, b_ref, w_ref, g2_ref):
    dv = dv_ref[...]
    z = jnp.maximum((s_ref[0] + s_ref[1] + g_ref[...]) * dv + b_ref[...], 0.0)
    h2 = jnp.dot(z, w_ref[...], preferred_element_type=jnp.float32)
    g2_ref[...] = h2 * dv


def _tc_mid(s, g1, dinv, b1, W2):
    return pl.pallas_call(
        _tc_mid_body,
        grid=(_GRID,),
        in_specs=[
            pl.BlockSpec((NC, _RB, N_HID), lambda i: (0, i, 0)),
            pl.BlockSpec((_RB, N_HID), lambda i: (i, 0)),
            pl.BlockSpec((_RB, 1), lambda i: (i, 0)),
            pl.BlockSpec((1, N_HID), lambda i: (0, 0)),
            pl.BlockSpec((N_HID, N_HID), lambda i: (0, 0)),
        ],
        out_specs=pl.BlockSpec((_RB, N_HID), lambda i: (i, 0)),
        out_shape=jax.ShapeDtypeStruct((N_NODES, N_HID), jnp.float32),
    )(s, g1, dinv, b1, W2)


def _tc_out_body(s_ref, g_ref, dv_ref, b_ref, o_ref):
    o_ref[...] = jnp.maximum(
        (s_ref[0] + s_ref[1] + g_ref[...]) * dv_ref[...] + b_ref[...], 0.0
    )


def _tc_out(s, g2, dinv, b2):
    return pl.pallas_call(
        _tc_out_body,
        grid=(_GRID,),
        in_specs=[
            pl.BlockSpec((NC, _RB, N_HID), lambda i: (0, i, 0)),
            pl.BlockSpec((_RB, N_HID), lambda i: (i, 0)),
            pl.BlockSpec((_RB, 1), lambda i: (i, 0)),
            pl.BlockSpec((1, N_HID), lambda i: (0, 0)),
        ],
        out_specs=pl.BlockSpec((_RB, N_HID), lambda i: (i, 0)),
        out_shape=jax.ShapeDtypeStruct((N_NODES, N_HID), jnp.float32),
    )(s, g2, dinv, b2)


@jax.jit
def kernel(x, edge_index, W1, b1, W2, b2):
    edge_flat = edge_index.astype(jnp.int32).reshape(2 * N_EDGES)

    h1 = _tc_mm(x, W1)
    degp = _sc_deg(edge_flat).reshape(NC, NPAD)
    deg0 = degp[0, :N_NODES].reshape(N_NODES, 1)
    deg1 = degp[1, :N_NODES].reshape(N_NODES, 1)

    g1, dinv = _tc_scale(h1, deg0, deg1)

    s = _sc_scatter(edge_flat, g1)
    g2 = _tc_mid(s, g1, dinv, b1.reshape(1, N_HID), W2)

    s2 = _sc_scatter(edge_flat, g2)
    return _tc_out(s2, g2, dinv, b2.reshape(1, N_HID))


# speedup vs baseline: 1.0708x; 1.0708x over previous
import functools

import jax
import jax.numpy as jnp
from jax import lax
from jax.experimental import pallas as pl
from jax.experimental.pallas import tpu as pltpu
from jax.experimental.pallas import tpu_sc as plsc

N_NODES = 10000
D_FEAT = 128
N_HID = 128
N_EDGES = 320000

NC = 2
NS = 16
NW = NC * NS
K = 128
NCHT = N_EDGES // K
CHB = NCHT // NW
CHX = NCHT - CHB * NW
NROWS = 10112
RPT = NROWS // NS
NPAD = 16384
DPT = NPAD // NS

_mesh = plsc.VectorSubcoreMesh(core_axis_name="c", subcore_axis_name="s")


def _chunk_range(c, t):
    w = c * NS + t
    start = w * CHB + jnp.minimum(w, CHX)
    nch = CHB + jnp.where(w < CHX, 1, 0)
    return start, nch


@functools.partial(
    pl.kernel,
    out_type=jax.ShapeDtypeStruct((NC * NPAD,), jnp.float32),
    mesh=_mesh,
    scratch_types=dict(
        deg_sh=pltpu.VMEM_SHARED((NPAD,), jnp.float32),
        dstb=pltpu.VMEM((CHB, K), jnp.int32),
        xtra=pltpu.VMEM((K,), jnp.int32),
        ones=pltpu.VMEM((K,), jnp.float32),
        zv=pltpu.VMEM((DPT,), jnp.float32),
        semd=pltpu.SemaphoreType.DMA,
    ),
)
def _sc_deg(edge_hbm, deg_out, *, deg_sh, dstb, xtra, ones, zv, semd):
    c = lax.axis_index("c")
    t = lax.axis_index("s")
    start, nch = _chunk_range(c, t)

    def stage(j, carry):
        off = pl.multiple_of(N_EDGES + (start + j) * K, 8)
        pltpu.async_copy(edge_hbm.at[pl.ds(off, K)], dstb.at[j], semd)
        return carry

    lax.fori_loop(0, CHB, stage, 0)

    @pl.when(nch > CHB)
    def _():
        pltpu.sync_copy(
            edge_hbm.at[pl.ds(pl.multiple_of(N_EDGES + (start + CHB) * K, 8), K)],
            xtra)

    def drain(j, carry):
        pltpu.make_async_copy(edge_hbm.at[pl.ds(0, K)], dstb.at[0], semd).wait()
        return carry

    lax.fori_loop(0, CHB, drain, 0)

    for i in range(K // 16):
        ones[pl.ds(i * 16, 16)] = jnp.ones((16,), jnp.float32)
    for i in range(DPT // 16):
        zv[pl.ds(i * 16, 16)] = jnp.zeros((16,), jnp.float32)
    pltpu.sync_copy(zv, deg_sh.at[pl.ds(t * DPT, DPT)])
    plsc.subcore_barrier()

    def chunk(j, carry):
        pltpu.sync_copy(ones, deg_sh.at[dstb.at[j]], add=True)
        return carry

    lax.fori_loop(0, CHB, chunk, 0)

    @pl.when(nch > CHB)
    def _():
        pltpu.sync_copy(ones, deg_sh.at[xtra], add=True)

    plsc.subcore_barrier()
    pltpu.sync_copy(
        deg_sh.at[pl.ds(t * DPT, DPT)],
        deg_out.at[pl.ds(c * NPAD + t * DPT, DPT)],
    )


@functools.partial(
    pl.kernel,
    out_type=jax.ShapeDtypeStruct((NC, NROWS, N_HID), jnp.float32),
    mesh=_mesh,
    scratch_types=dict(
        acc_sh=pltpu.VMEM_SHARED((NROWS, N_HID), jnp.float32),
        ib0s=pltpu.VMEM((K,), jnp.int32),
        ib1s=pltpu.VMEM((K,), jnp.int32),
        ib0d=pltpu.VMEM((K,), jnp.int32),
        ib1d=pltpu.VMEM((K,), jnp.int32),
        rows0=pltpu.VMEM((K, N_HID), jnp.float32),
        rows1=pltpu.VMEM((K, N_HID), jnp.float32),
        sem0=pltpu.SemaphoreType.DMA,
        sem1=pltpu.SemaphoreType.DMA,
        semi0s=pltpu.SemaphoreType.DMA,
        semi1s=pltpu.SemaphoreType.DMA,
        semi0d=pltpu.SemaphoreType.DMA,
        semi1d=pltpu.SemaphoreType.DMA,
    ),
)
def _sc_scatter(edge_hbm, g_hbm, out_hbm, *, acc_sh, ib0s, ib1s, ib0d, ib1d,
                rows0, rows1, sem0, sem1, semi0s, semi1s, semi0d, semi1d):
    c = lax.axis_index("c")
    t = lax.axis_index("s")
    start, nch = _chunk_range(c, t)

    def src_at(j):
        return edge_hbm.at[pl.ds(pl.multiple_of((start + j) * K, 8), K)]

    def dst_at(j):
        return edge_hbm.at[pl.ds(pl.multiple_of(N_EDGES + (start + j) * K, 8), K)]

    def zrow(i, carry):
        for j in range(N_HID // 16):
            rows0[i, pl.ds(j * 16, 16)] = jnp.zeros((16,), jnp.float32)
        return carry

    lax.fori_loop(0, K, zrow, 0)
    nz = RPT // K
    tail = RPT - nz * K
    for i in range(nz):
        pltpu.async_copy(rows0, acc_sh.at[pl.ds(t * RPT + i * K, K)], sem0)
    pltpu.async_copy(rows0.at[pl.ds(0, tail)],
                     acc_sh.at[pl.ds(t * RPT + nz * K, tail)], sem1)
    for _ in range(nz):
        pltpu.make_async_copy(rows0, acc_sh.at[pl.ds(t * RPT, K)], sem0).wait()
    pltpu.make_async_copy(rows0.at[pl.ds(0, tail)],
                          acc_sh.at[pl.ds(t * RPT, tail)], sem1).wait()
    plsc.subcore_barrier()

    pltpu.async_copy(src_at(0), ib0s, semi0s)
    pltpu.async_copy(src_at(1), ib1s, semi1s)
    pltpu.async_copy(dst_at(0), ib0d, semi0d)
    pltpu.async_copy(dst_at(1), ib1d, semi1d)
    pltpu.make_async_copy(src_at(0), ib0s, semi0s).wait()
    pltpu.async_copy(g_hbm.at[ib0s], rows0, sem0)
    pltpu.make_async_copy(src_at(1), ib1s, semi1s).wait()
    pltpu.async_copy(g_hbm.at[ib1s], rows1, sem1)

    def pair(i, carry):
        j0 = 2 * i
        pltpu.make_async_copy(g_hbm.at[ib0s], rows0, sem0).wait()
        pltpu.make_async_copy(dst_at(0), ib0d, semi0d).wait()

        @pl.when(j0 + 2 < nch)
        def _():
            pltpu.async_copy(src_at(j0 + 2), ib0s, semi0s)

        pltpu.sync_copy(rows0, acc_sh.at[ib0d], add=True)

        @pl.when(j0 + 2 < nch)
        def _():
            pltpu.async_copy(dst_at(j0 + 2), ib0d, semi0d)
            pltpu.make_async_copy(src_at(0), ib0s, semi0s).wait()
            pltpu.async_copy(g_hbm.at[ib0s], rows0, sem0)

        pltpu.make_async_copy(g_hbm.at[ib1s], rows1, sem1).wait()
        pltpu.make_async_copy(dst_at(1), ib1d, semi1d).wait()

        @pl.when(j0 + 3 < nch)
        def _():
            pltpu.async_copy(src_at(j0 + 3), ib1s, semi1s)

        pltpu.sync_copy(rows1, acc_sh.at[ib1d], add=True)

        @pl.when(j0 + 3 < nch)
        def _():
            pltpu.async_copy(dst_at(j0 + 3), ib1d, semi1d)
            pltpu.make_async_copy(src_at(0), ib1s, semi1s).wait()
            pltpu.async_copy(g_hbm.at[ib1s], rows1, sem1)

        return carry

    lax.fori_loop(0, CHB // 2, pair, 0)

    @pl.when(nch > CHB)
    def _():
        pltpu.make_async_copy(g_hbm.at[ib0s], rows0, sem0).wait()
        pltpu.make_async_copy(dst_at(0), ib0d, semi0d).wait()
        pltpu.sync_copy(rows0, acc_sh.at[ib0d], add=True)

    plsc.subcore_barrier()
    pltpu.sync_copy(
        acc_sh.at[pl.ds(t * RPT, RPT)],
        out_hbm.at[c, pl.ds(t * RPT, RPT)],
    )


_RB = 2048
_GB = _RB // 128
_GRID = 5
_MMRB = 2000
NG = 80


def _tc_mm_body(x_ref, w_ref, h_ref):
    h_ref[...] = jnp.dot(x_ref[...], w_ref[...],
                         preferred_element_type=jnp.float32)


def _tc_mm(x, W1):
    return pl.pallas_call(
        _tc_mm_body,
        grid=(N_NODES // _MMRB,),
        in_specs=[
            pl.BlockSpec((_MMRB, D_FEAT), lambda i: (i, 0)),
            pl.BlockSpec((D_FEAT, N_HID), lambda i: (0, 0)),
        ],
        out_specs=pl.BlockSpec((_MMRB, N_HID), lambda i: (i, 0)),
        out_shape=jax.ShapeDtypeStruct((N_NODES, N_HID), jnp.float32),
    )(x, W1)


def _tc_scale_body(h_ref, d_ref, g_ref, dinv_ref):
    deg = d_ref[0] + d_ref[1] + 1.0
    dv = lax.rsqrt(deg)
    dinv_ref[...] = dv
    h3 = h_ref[...].reshape(_GB, 128, N_HID)
    g_ref[...] = (h3 * dv[:, :, None]).reshape(_RB, N_HID)


def _tc_scale(h, degv):
    return pl.pallas_call(
        _tc_scale_body,
        grid=(_GRID,),
        in_specs=[
            pl.BlockSpec((_RB, N_HID), lambda i: (i, 0)),
            pl.BlockSpec((NC, _GB, 128), lambda i: (0, i, 0)),
        ],
        out_specs=[
            pl.BlockSpec((_RB, N_HID), lambda i: (i, 0)),
            pl.BlockSpec((_GB, 128), lambda i: (i, 0)),
        ],
        out_shape=[
            jax.ShapeDtypeStruct((_GRID * _RB, N_HID), jnp.float32),
            jax.ShapeDtypeStruct((NG, 128), jnp.float32),
        ],
    )(h, degv)


def _tc_mid_body(s_ref, g_ref, dv_ref, b_ref, w_ref, g2_ref):
    dv3 = dv_ref[...][:, :, None]
    agg = (s_ref[0] + s_ref[1] + g_ref[...]).reshape(_GB, 128, N_HID)
    z = jnp.maximum(agg * dv3 + b_ref[...].reshape(1, 1, N_HID), 0.0)
    h2 = jnp.dot(z.reshape(_RB, N_HID), w_ref[...],
                 preferred_element_type=jnp.float32)
    g2_ref[...] = (h2.reshape(_GB, 128, N_HID) * dv3).reshape(_RB, N_HID)


def _tc_mid(s, g1, dinv, b1, W2):
    return pl.pallas_call(
        _tc_mid_body,
        grid=(_GRID,),
        in_specs=[
            pl.BlockSpec((NC, _RB, N_HID), lambda i: (0, i, 0)),
            pl.BlockSpec((_RB, N_HID), lambda i: (i, 0)),
            pl.BlockSpec((_GB, 128), lambda i: (i, 0)),
            pl.BlockSpec((1, N_HID), lambda i: (0, 0)),
            pl.BlockSpec((N_HID, N_HID), lambda i: (0, 0)),
        ],
        out_specs=pl.BlockSpec((_RB, N_HID), lambda i: (i, 0)),
        out_shape=jax.ShapeDtypeStruct((_GRID * _RB, N_HID), jnp.float32),
    )(s, g1, dinv, b1, W2)


def _tc_out_body(s_ref, g_ref, dv_ref, b_ref, o_ref):
    dv3 = dv_ref[...][:, :, None]
    agg = (s_ref[0] + s_ref[1] + g_ref[...]).reshape(_GB, 128, N_HID)
    o_ref[...] = jnp.maximum(
        agg * dv3 + b_ref[...].reshape(1, 1, N_HID), 0.0
    ).reshape(_RB, N_HID)


def _tc_out(s, g2, dinv, b2):
    return pl.pallas_call(
        _tc_out_body,
        grid=(_GRID,),
        in_specs=[
            pl.BlockSpec((NC, _RB, N_HID), lambda i: (0, i, 0)),
            pl.BlockSpec((_RB, N_HID), lambda i: (i, 0)),
            pl.BlockSpec((_GB, 128), lambda i: (i, 0)),
            pl.BlockSpec((1, N_HID), lambda i: (0, 0)),
        ],
        out_specs=pl.BlockSpec((_RB, N_HID), lambda i: (i, 0)),
        out_shape=jax.ShapeDtypeStruct((N_NODES, N_HID), jnp.float32),
    )(s, g2, dinv, b2)


@jax.jit
def kernel(x, edge_index, W1, b1, W2, b2):
    edge_flat = edge_index.astype(jnp.int32).reshape(2 * N_EDGES)

    h1 = _tc_mm(x, W1)
    degv = _sc_deg(edge_flat).reshape(NC, NPAD // 128, 128)

    g1, dinv = _tc_scale(h1, degv)

    s = _sc_scatter(edge_flat, g1)
    g2 = _tc_mid(s, g1, dinv, b1.reshape(1, N_HID), W2)

    s2 = _sc_scatter(edge_flat, g2)
    return _tc_out(s2, g2, dinv, b2.reshape(1, N_HID))


# speedup vs baseline: 1.0913x; 1.0191x over previous
import functools

import jax
import jax.numpy as jnp
from jax import lax
from jax.experimental import pallas as pl
from jax.experimental.pallas import tpu as pltpu
from jax.experimental.pallas import tpu_sc as plsc

N_NODES = 10000
D_FEAT = 128
N_HID = 128
N_EDGES = 320000

NC = 2
NS = 16
NW = NC * NS
K = 128
NCHT = N_EDGES // K
CHB = NCHT // NW
CHX = NCHT - CHB * NW
NROWS = 10112
RPT = NROWS // NS
NPAD = 16384
DPT = NPAD // NS

_mesh = plsc.VectorSubcoreMesh(core_axis_name="c", subcore_axis_name="s")


def _chunk_range(c, t):
    w = c * NS + t
    start = w * CHB + jnp.minimum(w, CHX)
    nch = CHB + jnp.where(w < CHX, 1, 0)
    return start, nch


@functools.partial(
    pl.kernel,
    out_type=jax.ShapeDtypeStruct((NC * NPAD,), jnp.float32),
    mesh=_mesh,
    scratch_types=dict(
        deg_sh=pltpu.VMEM_SHARED((NPAD,), jnp.float32),
        dstb=pltpu.VMEM((CHB, K), jnp.int32),
        xtra=pltpu.VMEM((K,), jnp.int32),
        ones=pltpu.VMEM((K,), jnp.float32),
        zv=pltpu.VMEM((DPT,), jnp.float32),
        semd=pltpu.SemaphoreType.DMA,
    ),
)
def _sc_deg(edge_hbm, deg_out, *, deg_sh, dstb, xtra, ones, zv, semd):
    c = lax.axis_index("c")
    t = lax.axis_index("s")
    start, nch = _chunk_range(c, t)

    def stage(j, carry):
        off = pl.multiple_of(N_EDGES + (start + j) * K, 8)
        pltpu.async_copy(edge_hbm.at[pl.ds(off, K)], dstb.at[j], semd)
        return carry

    lax.fori_loop(0, CHB, stage, 0)

    @pl.when(nch > CHB)
    def _():
        pltpu.sync_copy(
            edge_hbm.at[pl.ds(pl.multiple_of(N_EDGES + (start + CHB) * K, 8), K)],
            xtra)

    def drain(j, carry):
        pltpu.make_async_copy(edge_hbm.at[pl.ds(0, K)], dstb.at[0], semd).wait()
        return carry

    lax.fori_loop(0, CHB, drain, 0)

    for i in range(K // 16):
        ones[pl.ds(i * 16, 16)] = jnp.ones((16,), jnp.float32)
    for i in range(DPT // 16):
        zv[pl.ds(i * 16, 16)] = jnp.zeros((16,), jnp.float32)
    pltpu.sync_copy(zv, deg_sh.at[pl.ds(t * DPT, DPT)])
    plsc.subcore_barrier()

    def chunk(j, carry):
        pltpu.async_copy(ones, deg_sh.at[dstb.at[j]], semd, add=True)
        return carry

    lax.fori_loop(0, CHB, chunk, 0)

    @pl.when(nch > CHB)
    def _():
        pltpu.async_copy(ones, deg_sh.at[xtra], semd, add=True)

    def dchunk(j, carry):
        pltpu.make_async_copy(ones, deg_sh.at[dstb.at[0]], semd).wait()
        return carry

    lax.fori_loop(0, nch, dchunk, 0)
    plsc.subcore_barrier()
    pltpu.sync_copy(
        deg_sh.at[pl.ds(t * DPT, DPT)],
        deg_out.at[pl.ds(c * NPAD + t * DPT, DPT)],
    )


@functools.partial(
    pl.kernel,
    out_type=jax.ShapeDtypeStruct((NC, NROWS, N_HID), jnp.float32),
    mesh=_mesh,
    scratch_types=dict(
        acc_sh=pltpu.VMEM_SHARED((NROWS, N_HID), jnp.float32),
        ib0s=pltpu.VMEM((K,), jnp.int32),
        ib1s=pltpu.VMEM((K,), jnp.int32),
        ib0d=pltpu.VMEM((K,), jnp.int32),
        ib1d=pltpu.VMEM((K,), jnp.int32),
        rows0=pltpu.VMEM((K, N_HID), jnp.float32),
        rows1=pltpu.VMEM((K, N_HID), jnp.float32),
        sem0=pltpu.SemaphoreType.DMA,
        sem1=pltpu.SemaphoreType.DMA,
        semi0s=pltpu.SemaphoreType.DMA,
        semi1s=pltpu.SemaphoreType.DMA,
        semi0d=pltpu.SemaphoreType.DMA,
        semi1d=pltpu.SemaphoreType.DMA,
    ),
)
def _sc_scatter(edge_hbm, g_hbm, out_hbm, *, acc_sh, ib0s, ib1s, ib0d, ib1d,
                rows0, rows1, sem0, sem1, semi0s, semi1s, semi0d, semi1d):
    c = lax.axis_index("c")
    t = lax.axis_index("s")
    start, nch = _chunk_range(c, t)

    def src_at(j):
        return edge_hbm.at[pl.ds(pl.multiple_of((start + j) * K, 8), K)]

    def dst_at(j):
        return edge_hbm.at[pl.ds(pl.multiple_of(N_EDGES + (start + j) * K, 8), K)]

    def zrow(i, carry):
        for j in range(N_HID // 16):
            rows0[i, pl.ds(j * 16, 16)] = jnp.zeros((16,), jnp.float32)
        return carry

    lax.fori_loop(0, K, zrow, 0)
    nz = RPT // K
    tail = RPT - nz * K
    for i in range(nz):
        pltpu.async_copy(rows0, acc_sh.at[pl.ds(t * RPT + i * K, K)], sem0)
    pltpu.async_copy(rows0.at[pl.ds(0, tail)],
                     acc_sh.at[pl.ds(t * RPT + nz * K, tail)], sem1)
    for _ in range(nz):
        pltpu.make_async_copy(rows0, acc_sh.at[pl.ds(t * RPT, K)], sem0).wait()
    pltpu.make_async_copy(rows0.at[pl.ds(0, tail)],
                          acc_sh.at[pl.ds(t * RPT, tail)], sem1).wait()
    plsc.subcore_barrier()

    pltpu.async_copy(src_at(0), ib0s, semi0s)
    pltpu.async_copy(src_at(1), ib1s, semi1s)
    pltpu.async_copy(dst_at(0), ib0d, semi0d)
    pltpu.async_copy(dst_at(1), ib1d, semi1d)
    pltpu.make_async_copy(src_at(0), ib0s, semi0s).wait()
    pltpu.async_copy(g_hbm.at[ib0s], rows0, sem0)
    pltpu.make_async_copy(src_at(1), ib1s, semi1s).wait()
    pltpu.async_copy(g_hbm.at[ib1s], rows1, sem1)

    def pair(i, carry):
        j0 = 2 * i
        pltpu.make_async_copy(g_hbm.at[ib0s], rows0, sem0).wait()
        pltpu.make_async_copy(dst_at(0), ib0d, semi0d).wait()

        @pl.when(j0 + 2 < nch)
        def _():
            pltpu.async_copy(src_at(j0 + 2), ib0s, semi0s)

        pltpu.sync_copy(rows0, acc_sh.at[ib0d], add=True)

        @pl.when(j0 + 2 < nch)
        def _():
            pltpu.async_copy(dst_at(j0 + 2), ib0d, semi0d)
            pltpu.make_async_copy(src_at(0), ib0s, semi0s).wait()
            pltpu.async_copy(g_hbm.at[ib0s], rows0, sem0)

        pltpu.make_async_copy(g_hbm.at[ib1s], rows1, sem1).wait()
        pltpu.make_async_copy(dst_at(1), ib1d, semi1d).wait()

        @pl.when(j0 + 3 < nch)
        def _():
            pltpu.async_copy(src_at(j0 + 3), ib1s, semi1s)

        pltpu.sync_copy(rows1, acc_sh.at[ib1d], add=True)

        @pl.when(j0 + 3 < nch)
        def _():
            pltpu.async_copy(dst_at(j0 + 3), ib1d, semi1d)
            pltpu.make_async_copy(src_at(0), ib1s, semi1s).wait()
            pltpu.async_copy(g_hbm.at[ib1s], rows1, sem1)

        return carry

    lax.fori_loop(0, CHB // 2, pair, 0)

    @pl.when(nch > CHB)
    def _():
        pltpu.make_async_copy(g_hbm.at[ib0s], rows0, sem0).wait()
        pltpu.make_async_copy(dst_at(0), ib0d, semi0d).wait()
        pltpu.sync_copy(rows0, acc_sh.at[ib0d], add=True)

    plsc.subcore_barrier()
    pltpu.sync_copy(
        acc_sh.at[pl.ds(t * RPT, RPT)],
        out_hbm.at[c, pl.ds(t * RPT, RPT)],
    )


_RB = 2048
_GB = _RB // 128
_GRID = 5
_MMRB = 2000
NG = 80


def _tc_mm_body(x_ref, w_ref, h_ref):
    h_ref[...] = jnp.dot(x_ref[...], w_ref[...],
                         preferred_element_type=jnp.float32)


def _tc_mm(x, W1):
    return pl.pallas_call(
        _tc_mm_body,
        grid=(N_NODES // _MMRB,),
        in_specs=[
            pl.BlockSpec((_MMRB, D_FEAT), lambda i: (i, 0)),
            pl.BlockSpec((D_FEAT, N_HID), lambda i: (0, 0)),
        ],
        out_specs=pl.BlockSpec((_MMRB, N_HID), lambda i: (i, 0)),
        out_shape=jax.ShapeDtypeStruct((N_NODES, N_HID), jnp.float32),
    )(x, W1)


def _tc_scale_body(h_ref, d_ref, g_ref, dinv_ref):
    deg = d_ref[0] + d_ref[1] + 1.0
    dv = lax.rsqrt(deg)
    dinv_ref[...] = dv
    h3 = h_ref[...].reshape(_GB, 128, N_HID)
    g_ref[...] = (h3 * dv[:, :, None]).reshape(_RB, N_HID)


def _tc_scale(h, degv):
    return pl.pallas_call(
        _tc_scale_body,
        grid=(_GRID,),
        in_specs=[
            pl.BlockSpec((_RB, N_HID), lambda i: (i, 0)),
            pl.BlockSpec((NC, _GB, 128), lambda i: (0, i, 0)),
        ],
        out_specs=[
            pl.BlockSpec((_RB, N_HID), lambda i: (i, 0)),
            pl.BlockSpec((_GB, 128), lambda i: (i, 0)),
        ],
        out_shape=[
            jax.ShapeDtypeStruct((_GRID * _RB, N_HID), jnp.float32),
            jax.ShapeDtypeStruct((NG, 128), jnp.float32),
        ],
    )(h, degv)


def _tc_mid_body(s_ref, g_ref, dv_ref, b_ref, w_ref, g2_ref):
    dv3 = dv_ref[...][:, :, None]
    agg = (s_ref[0] + s_ref[1] + g_ref[...]).reshape(_GB, 128, N_HID)
    z = jnp.maximum(agg * dv3 + b_ref[...].reshape(1, 1, N_HID), 0.0)
    h2 = jnp.dot(z.reshape(_RB, N_HID), w_ref[...],
                 preferred_element_type=jnp.float32)
    g2_ref[...] = (h2.reshape(_GB, 128, N_HID) * dv3).reshape(_RB, N_HID)


def _tc_mid(s, g1, dinv, b1, W2):
    return pl.pallas_call(
        _tc_mid_body,
        grid=(_GRID,),
        in_specs=[
            pl.BlockSpec((NC, _RB, N_HID), lambda i: (0, i, 0)),
            pl.BlockSpec((_RB, N_HID), lambda i: (i, 0)),
            pl.BlockSpec((_GB, 128), lambda i: (i, 0)),
            pl.BlockSpec((1, N_HID), lambda i: (0, 0)),
            pl.BlockSpec((N_HID, N_HID), lambda i: (0, 0)),
        ],
        out_specs=pl.BlockSpec((_RB, N_HID), lambda i: (i, 0)),
        out_shape=jax.ShapeDtypeStruct((_GRID * _RB, N_HID), jnp.float32),
    )(s, g1, dinv, b1, W2)


def _tc_out_body(s_ref, g_ref, dv_ref, b_ref, o_ref):
    dv3 = dv_ref[...][:, :, None]
    agg = (s_ref[0] + s_ref[1] + g_ref[...]).reshape(_GB, 128, N_HID)
    o_ref[...] = jnp.maximum(
        agg * dv3 + b_ref[...].reshape(1, 1, N_HID), 0.0
    ).reshape(_RB, N_HID)


def _tc_out(s, g2, dinv, b2):
    return pl.pallas_call(
        _tc_out_body,
        grid=(_GRID,),
        in_specs=[
            pl.BlockSpec((NC, _RB, N_HID), lambda i: (0, i, 0)),
            pl.BlockSpec((_RB, N_HID), lambda i: (i, 0)),
            pl.BlockSpec((_GB, 128), lambda i: (i, 0)),
            pl.BlockSpec((1, N_HID), lambda i: (0, 0)),
        ],
        out_specs=pl.BlockSpec((_RB, N_HID), lambda i: (i, 0)),
        out_shape=jax.ShapeDtypeStruct((N_NODES, N_HID), jnp.float32),
    )(s, g2, dinv, b2)


@jax.jit
def kernel(x, edge_index, W1, b1, W2, b2):
    edge_flat = edge_index.astype(jnp.int32).reshape(2 * N_EDGES)

    h1 = _tc_mm(x, W1)
    degv = _sc_deg(edge_flat).reshape(NC, NPAD // 128, 128)

    g1, dinv = _tc_scale(h1, degv)

    s = _sc_scatter(edge_flat, g1)
    g2 = _tc_mid(s, g1, dinv, b1.reshape(1, N_HID), W2)

    s2 = _sc_scatter(edge_flat, g2)
    return _tc_out(s2, g2, dinv, b2.reshape(1, N_HID))
